# baseline (speedup 1.0000x reference)
import functools

import jax
import jax.numpy as jnp
from jax import lax
from jax.experimental import pallas as pl
from jax.experimental.pallas import tpu as pltpu

N_OBJ = 10000
N_EDGE = 160000
N_CLS = 151
HID = 256
N_LAYERS = 4
ROWT = 1000
GRID = N_OBJ // ROWT


def _enc_body(box_f_ref, w1_ref, b1_ref, g_ref, bb_ref, w2_ref, b2_ref,
              x_ref, lg_ref, box_t_ref, oew_ref, wcx_ref, wce_ref, wcp_ref,
              bctx_ref, wout_ref, bout_ref,
              repa_ref, repb_ref, preds_ref):
    h = box_f_ref[...] @ w1_ref[...] + b1_ref[...]
    mu = jnp.mean(h, axis=0, keepdims=True)
    var = jnp.mean((h - mu) ** 2, axis=0, keepdims=True)
    scale = g_ref[...] * lax.rsqrt(var + 1e-5)
    shift = bb_ref[...] - mu * scale
    ht = box_t_ref[...] @ w1_ref[...] + b1_ref[...]
    pos = jnp.maximum((ht * scale + shift) @ w2_ref[...] + b2_ref[...], 0.0)
    lg = lg_ref[...]
    p = jax.nn.softmax(lg, axis=-1)
    e = p @ oew_ref[...]
    rep = (x_ref[...] @ wcx_ref[...] + e @ wce_ref[...] + pos @ wcp_ref[...]
           + bctx_ref[...])
    d0 = rep @ wout_ref[...] + bout_ref[...]
    col = lax.broadcasted_iota(jnp.int32, d0.shape, 1)
    d0m = jnp.where(col > 0, d0, -jnp.inf)
    pr = jnp.argmax(d0m, axis=-1).astype(jnp.int32)
    repa_ref[...] = rep[:, :128]
    repb_ref[...] = rep[:, 128:]
    preds_ref[...] = pr[:, None]


def _encoder(box_info, pos_w1, pos_b1, pos_bn_g, pos_bn_b, pos_w2, pos_b2,
             x, obj_logits, obj_embed_w, W_ctx, b_ctx, W_out, b_out):
    full = lambda shape: pl.BlockSpec(shape, lambda i: (0, 0))
    tile = lambda cols: pl.BlockSpec((ROWT, cols), lambda i: (i, 0))
    out = pl.pallas_call(
        _enc_body,
        grid=(GRID,),
        in_specs=[
            full((N_OBJ, 9)), full((9, 32)), full((1, 32)), full((1, 32)),
            full((1, 32)), full((32, 128)), full((1, 128)),
            tile(128), tile(N_CLS), tile(9), full((N_CLS, 200)),
            full((128, HID)), full((200, HID)), full((128, HID)),
            full((1, HID)), full((HID, N_CLS)), full((1, N_CLS)),
        ],
        out_specs=[tile(128), tile(128),
                   pl.BlockSpec((ROWT, 1), lambda i: (i, 0))],
        out_shape=[
            jax.ShapeDtypeStruct((N_OBJ, 128), jnp.float32),
            jax.ShapeDtypeStruct((N_OBJ, 128), jnp.float32),
            jax.ShapeDtypeStruct((N_OBJ, 1), jnp.int32),
        ],
    )(box_info, pos_w1, pos_b1.reshape(1, -1), pos_bn_g.reshape(1, -1),
      pos_bn_b.reshape(1, -1), pos_w2, pos_b2.reshape(1, -1),
      x, obj_logits, box_info, obj_embed_w,
      W_ctx[:128], W_ctx[128:328], W_ctx[328:], b_ctx.reshape(1, -1),
      W_out, b_out.reshape(1, -1))
    repa, repb, preds2 = out
    return repa, repb, preds2[:, 0]


def _tab_body(fb_ref, out_ref):
    fb = fb_ref[...]
    m = jnp.max(fb, axis=-1, keepdims=True)
    s = jnp.sum(jnp.exp(fb - m), axis=-1, keepdims=True)
    out_ref[...] = 1.0 / s


def _pair_table(freq_bias):
    out = pl.pallas_call(
        _tab_body,
        out_shape=jax.ShapeDtypeStruct((N_CLS * N_CLS, 1), jnp.float32),
    )(freq_bias)
    return out.reshape(N_CLS, N_CLS)


def _upd_body(agga_ref, aggb_ref, repa_ref, repb_ref, wt_ref, wb_ref,
              newa_ref, newb_ref):
    o = jnp.maximum(agga_ref[...] @ wt_ref[...] + aggb_ref[...] @ wb_ref[...],
                    0.0)
    newa_ref[...] = o[:, :128] + repa_ref[...]
    newb_ref[...] = o[:, 128:] + repb_ref[...]


def _update(agga, aggb, repa, repb, wm):
    tile = pl.BlockSpec((ROWT, 128), lambda i: (i, 0))
    full = lambda shape: pl.BlockSpec(shape, lambda i: (0, 0))
    return pl.pallas_call(
        _upd_body,
        grid=(GRID,),
        in_specs=[tile, tile, tile, tile, full((128, HID)), full((128, HID))],
        out_specs=[tile, tile],
        out_shape=[jax.ShapeDtypeStruct((N_OBJ, 128), jnp.float32)] * 2,
    )(agga, aggb, repa, repb, wm[:128], wm[128:])


def _final_body(agga_ref, aggb_ref, repa_ref, repb_ref, wt_ref, wb_ref,
                wouta_ref, woutb_ref, bout_ref, dists_ref):
    o = jnp.maximum(agga_ref[...] @ wt_ref[...] + aggb_ref[...] @ wb_ref[...],
                    0.0)
    na = o[:, :128] + repa_ref[...]
    nb = o[:, 128:] + repb_ref[...]
    dists_ref[...] = na @ wouta_ref[...] + nb @ woutb_ref[...] + bout_ref[...]


def _final(agga, aggb, repa, repb, wm, W_out, b_out):
    tile = pl.BlockSpec((ROWT, 128), lambda i: (i, 0))
    full = lambda shape: pl.BlockSpec(shape, lambda i: (0, 0))
    return pl.pallas_call(
        _final_body,
        grid=(GRID,),
        in_specs=[tile, tile, tile, tile, full((128, HID)), full((128, HID)),
                  full((128, N_CLS)), full((128, N_CLS)), full((1, N_CLS))],
        out_specs=pl.BlockSpec((ROWT, N_CLS), lambda i: (i, 0)),
        out_shape=jax.ShapeDtypeStruct((N_OBJ, N_CLS), jnp.float32),
    )(agga, aggb, repa, repb, wm[:128], wm[128:],
      W_out[:128], W_out[128:], b_out.reshape(1, -1))


def kernel(x, obj_logits, box_info, rel_pair_idxs, freq_bias, obj_embed_w,
           pos_w1, pos_b1, pos_bn_g, pos_bn_b, pos_w2, pos_b2,
           W_ctx, b_ctx, W_out, b_out, W_lin, b_lin, W_msg):
    repa, repb, preds = _encoder(box_info, pos_w1, pos_b1, pos_bn_g,
                                 pos_bn_b, pos_w2, pos_b2, x, obj_logits,
                                 obj_embed_w, W_ctx, b_ctx, W_out, b_out)
    tabw = _pair_table(freq_bias)
    src = rel_pair_idxs[0]
    dst = rel_pair_idxs[1]
    w = tabw[preds[src], preds[dst]]
    for i in range(N_LAYERS):
        rep = jnp.concatenate([repa, repb], axis=-1)
        msg = rep[src] * w[:, None]
        agg = jax.ops.segment_sum(msg, dst, num_segments=N_OBJ)
        if i < N_LAYERS - 1:
            repa, repb = _update(agg[:, :128], agg[:, 128:], repa, repb,
                                 W_msg[i])
        else:
            return _final(agg[:, :128], agg[:, 128:], repa, repb, W_msg[i],
                          W_out, b_out)


# speedup vs baseline: 4.8412x; 4.8412x over previous
import functools

import jax
import jax.numpy as jnp
from jax import lax
from jax.experimental import pallas as pl
from jax.experimental.pallas import tpu as pltpu
from jax.experimental.pallas import tpu_sc as plsc

N_OBJ = 10000
N_EDGE = 160000
N_CLS = 151
HID = 256
N_LAYERS = 4
ROWT = 1000
GRID = N_OBJ // ROWT

_NC = 2
_NS = 16
_L = 16
_NW = _NC * _NS
_EPW = 5008
_EPAD = _EPW * _NW

_SC_MESH = dict(core_axis_name="c", subcore_axis_name="s",
                num_cores=_NC, num_subcores=_NS)


def _edge_w_body(preds_hbm, tab_hbm, src_hbm, dst_hbm, w_hbm,
                 preds_v, tab_v, src_v, dst_v, w_v):
    c = lax.axis_index("c")
    s = lax.axis_index("s")
    wid = s * _NC + c
    base = wid * _EPW
    pltpu.sync_copy(preds_hbm, preds_v)
    pltpu.sync_copy(tab_hbm, tab_v)
    pltpu.sync_copy(src_hbm.at[pl.ds(base, _EPW)], src_v)
    pltpu.sync_copy(dst_hbm.at[pl.ds(base, _EPW)], dst_v)

    def body(i, carry):
        sl = pl.ds(i * _L, _L)
        ps = plsc.load_gather(preds_v, [src_v[sl]])
        pd = plsc.load_gather(preds_v, [dst_v[sl]])
        w_v[sl] = plsc.load_gather(tab_v, [ps, pd])
        return carry

    lax.fori_loop(0, _EPW // _L, body, 0)
    pltpu.sync_copy(w_v, w_hbm.at[pl.ds(base, _EPW)])


def _edge_weights(preds, tabw, src_pad, dst_pad):
    k = pl.kernel(
        _edge_w_body,
        out_type=jax.ShapeDtypeStruct((_EPAD,), jnp.float32),
        mesh=plsc.VectorSubcoreMesh(**_SC_MESH),
        compiler_params=pltpu.CompilerParams(needs_layout_passes=False),
        scratch_types=[
            pltpu.VMEM((N_OBJ,), jnp.int32),
            pltpu.VMEM((N_CLS, N_CLS), jnp.float32),
            pltpu.VMEM((_EPW,), jnp.int32),
            pltpu.VMEM((_EPW,), jnp.int32),
            pltpu.VMEM((_EPW,), jnp.float32),
        ],
    )
    return k(preds, tabw, src_pad, dst_pad)


def _enc_body(box_f_ref, w1_ref, b1_ref, g_ref, bb_ref, w2_ref, b2_ref,
              x_ref, lg_ref, box_t_ref, oew_ref, wcx_ref, wce_ref, wcp_ref,
              bctx_ref, wout_ref, bout_ref,
              repa_ref, repb_ref, preds_ref):
    h = box_f_ref[...] @ w1_ref[...] + b1_ref[...]
    mu = jnp.mean(h, axis=0, keepdims=True)
    var = jnp.mean((h - mu) ** 2, axis=0, keepdims=True)
    scale = g_ref[...] * lax.rsqrt(var + 1e-5)
    shift = bb_ref[...] - mu * scale
    ht = box_t_ref[...] @ w1_ref[...] + b1_ref[...]
    pos = jnp.maximum((ht * scale + shift) @ w2_ref[...] + b2_ref[...], 0.0)
    lg = lg_ref[...]
    p = jax.nn.softmax(lg, axis=-1)
    e = p @ oew_ref[...]
    rep = (x_ref[...] @ wcx_ref[...] + e @ wce_ref[...] + pos @ wcp_ref[...]
           + bctx_ref[...])
    d0 = rep @ wout_ref[...] + bout_ref[...]
    col = lax.broadcasted_iota(jnp.int32, d0.shape, 1)
    d0m = jnp.where(col > 0, d0, -jnp.inf)
    pr = jnp.argmax(d0m, axis=-1).astype(jnp.int32)
    repa_ref[...] = rep[:, :128]
    repb_ref[...] = rep[:, 128:]
    preds_ref[...] = pr[:, None]


def _encoder(box_info, pos_w1, pos_b1, pos_bn_g, pos_bn_b, pos_w2, pos_b2,
             x, obj_logits, obj_embed_w, W_ctx, b_ctx, W_out, b_out):
    full = lambda shape: pl.BlockSpec(shape, lambda i: (0, 0))
    tile = lambda cols: pl.BlockSpec((ROWT, cols), lambda i: (i, 0))
    out = pl.pallas_call(
        _enc_body,
        grid=(GRID,),
        in_specs=[
            full((N_OBJ, 9)), full((9, 32)), full((1, 32)), full((1, 32)),
            full((1, 32)), full((32, 128)), full((1, 128)),
            tile(128), tile(N_CLS), tile(9), full((N_CLS, 200)),
            full((128, HID)), full((200, HID)), full((128, HID)),
            full((1, HID)), full((HID, N_CLS)), full((1, N_CLS)),
        ],
        out_specs=[tile(128), tile(128),
                   pl.BlockSpec((ROWT, 1), lambda i: (i, 0))],
        out_shape=[
            jax.ShapeDtypeStruct((N_OBJ, 128), jnp.float32),
            jax.ShapeDtypeStruct((N_OBJ, 128), jnp.float32),
            jax.ShapeDtypeStruct((N_OBJ, 1), jnp.int32),
        ],
    )(box_info, pos_w1, pos_b1.reshape(1, -1), pos_bn_g.reshape(1, -1),
      pos_bn_b.reshape(1, -1), pos_w2, pos_b2.reshape(1, -1),
      x, obj_logits, box_info, obj_embed_w,
      W_ctx[:128], W_ctx[128:328], W_ctx[328:], b_ctx.reshape(1, -1),
      W_out, b_out.reshape(1, -1))
    repa, repb, preds2 = out
    return repa, repb, preds2[:, 0]


_CH = 80
_NCHUNK = 125
_NPAD = 10240
_RPT = _NPAD // _NS
_DR = 128


def _agg_body(repa_hbm, repb_hbm, pk_hbm, agga_hbm, aggb_hbm,
              pack_c, rows, agg_sh):
    c = lax.axis_index("c")
    s = lax.axis_index("s")

    def zrow(j, carry):
        for r in range(8):
            rows[j, pl.ds(r * _L, _L)] = jnp.zeros((_L,), jnp.float32)
        return carry
    lax.fori_loop(0, _CH, zrow, 0)

    def zcopy(k2, carry):
        pltpu.sync_copy(rows, agg_sh.at[pl.ds(s * _RPT + k2 * _CH, _CH)])
        return carry
    lax.fori_loop(0, _RPT // _CH, zcopy, 0)
    plsc.subcore_barrier()

    def chunk(i, carry):
        pltpu.sync_copy(pk_hbm.at[s, i], pack_c)

        @pl.when(c == 0)
        def _():
            pltpu.sync_copy(repa_hbm.at[pack_c.at[0]], rows)

        @pl.when(c == 1)
        def _():
            pltpu.sync_copy(repb_hbm.at[pack_c.at[0]], rows)

        two = jnp.full((_L,), 2, jnp.int32)

        def scale(e, carry2):
            wv = plsc.bitcast(
                plsc.load_gather(pack_c, [two, jnp.full((_L,), e, jnp.int32)]),
                jnp.float32)
            for r in range(8):
                sl = pl.ds(r * _L, _L)
                rows[e, sl] = rows[e, sl] * wv
            return carry2
        lax.fori_loop(0, _CH, scale, 0)
        pltpu.sync_copy(rows, agg_sh.at[pack_c.at[1]], add=True)
        return carry
    lax.fori_loop(0, _NCHUNK, chunk, 0)
    plsc.subcore_barrier()

    def drain(k2, carry):
        off = s * _RPT + k2 * _CH
        pltpu.sync_copy(agg_sh.at[pl.ds(off, _CH)], rows)

        @pl.when(c == 0)
        def _():
            pltpu.sync_copy(rows, agga_hbm.at[pl.ds(off, _CH)])

        @pl.when(c == 1)
        def _():
            pltpu.sync_copy(rows, aggb_hbm.at[pl.ds(off, _CH)])
        return carry
    lax.fori_loop(0, _RPT // _CH, drain, 0)


def _aggregate(repa, repb, packed):
    k = pl.kernel(
        _agg_body,
        out_type=[jax.ShapeDtypeStruct((_NPAD, 128), jnp.float32)] * 2,
        mesh=plsc.VectorSubcoreMesh(**_SC_MESH),
        compiler_params=pltpu.CompilerParams(needs_layout_passes=False),
        scratch_types=[
            pltpu.VMEM((3, _CH), jnp.int32),
            pltpu.VMEM((_CH, 128), jnp.float32),
            pltpu.VMEM_SHARED((_NPAD, 128), jnp.float32),
        ],
    )
    return k(repa, repb, packed)


def _tab_body(fb_ref, out_ref):
    fb = fb_ref[...]
    m = jnp.max(fb, axis=-1, keepdims=True)
    s = jnp.sum(jnp.exp(fb - m), axis=-1, keepdims=True)
    out_ref[...] = 1.0 / s


def _pair_table(freq_bias):
    out = pl.pallas_call(
        _tab_body,
        out_shape=jax.ShapeDtypeStruct((N_CLS * N_CLS, 1), jnp.float32),
    )(freq_bias)
    return out.reshape(N_CLS, N_CLS)


def _upd_body(agga_ref, aggb_ref, repa_ref, repb_ref, wt_ref, wb_ref,
              newa_ref, newb_ref):
    o = jnp.maximum(agga_ref[...] @ wt_ref[...] + aggb_ref[...] @ wb_ref[...],
                    0.0)
    newa_ref[...] = o[:, :128] + repa_ref[...]
    newb_ref[...] = o[:, 128:] + repb_ref[...]


def _update(agga, aggb, repa, repb, wm):
    tile = pl.BlockSpec((ROWT, 128), lambda i: (i, 0))
    full = lambda shape: pl.BlockSpec(shape, lambda i: (0, 0))
    return pl.pallas_call(
        _upd_body,
        grid=(GRID,),
        in_specs=[tile, tile, tile, tile, full((128, HID)), full((128, HID))],
        out_specs=[tile, tile],
        out_shape=[jax.ShapeDtypeStruct((N_OBJ, 128), jnp.float32)] * 2,
    )(agga, aggb, repa, repb, wm[:128], wm[128:])


def _final_body(agga_ref, aggb_ref, repa_ref, repb_ref, wt_ref, wb_ref,
                wouta_ref, woutb_ref, bout_ref, dists_ref):
    o = jnp.maximum(agga_ref[...] @ wt_ref[...] + aggb_ref[...] @ wb_ref[...],
                    0.0)
    na = o[:, :128] + repa_ref[...]
    nb = o[:, 128:] + repb_ref[...]
    dists_ref[...] = na @ wouta_ref[...] + nb @ woutb_ref[...] + bout_ref[...]


def _final(agga, aggb, repa, repb, wm, W_out, b_out):
    tile = pl.BlockSpec((ROWT, 128), lambda i: (i, 0))
    full = lambda shape: pl.BlockSpec(shape, lambda i: (0, 0))
    return pl.pallas_call(
        _final_body,
        grid=(GRID,),
        in_specs=[tile, tile, tile, tile, full((128, HID)), full((128, HID)),
                  full((128, N_CLS)), full((128, N_CLS)), full((1, N_CLS))],
        out_specs=pl.BlockSpec((ROWT, N_CLS), lambda i: (i, 0)),
        out_shape=jax.ShapeDtypeStruct((N_OBJ, N_CLS), jnp.float32),
    )(agga, aggb, repa, repb, wm[:128], wm[128:],
      W_out[:128], W_out[128:], b_out.reshape(1, -1))


def kernel(x, obj_logits, box_info, rel_pair_idxs, freq_bias, obj_embed_w,
           pos_w1, pos_b1, pos_bn_g, pos_bn_b, pos_w2, pos_b2,
           W_ctx, b_ctx, W_out, b_out, W_lin, b_lin, W_msg):
    repa, repb, preds = _encoder(box_info, pos_w1, pos_b1, pos_bn_g,
                                 pos_bn_b, pos_w2, pos_b2, x, obj_logits,
                                 obj_embed_w, W_ctx, b_ctx, W_out, b_out)
    tabw = _pair_table(freq_bias)
    src = rel_pair_idxs[0]
    dst = rel_pair_idxs[1]
    pad = jnp.zeros((_EPAD - N_EDGE,), jnp.int32)
    src_pad = jnp.concatenate([src.astype(jnp.int32), pad])
    dst_pad = jnp.concatenate([dst.astype(jnp.int32), pad])
    w = _edge_weights(preds, tabw, src_pad, dst_pad)[:N_EDGE]
    src3 = src.astype(jnp.int32).reshape(_NS, _NCHUNK, 1, _CH)
    dst3 = dst.astype(jnp.int32).reshape(_NS, _NCHUNK, 1, _CH)
    w3 = lax.bitcast_convert_type(w, jnp.int32).reshape(_NS, _NCHUNK, 1, _CH)
    packed = jnp.concatenate([src3, dst3, w3], axis=2)
    for i in range(N_LAYERS):
        agga, aggb = _aggregate(repa, repb, packed)
        agga = agga[:N_OBJ]
        aggb = aggb[:N_OBJ]
        if i < N_LAYERS - 1:
            repa, repb = _update(agga, aggb, repa, repb, W_msg[i])
        else:
            return _final(agga, aggb, repa, repb, W_msg[i], W_out, b_out)
